# baseline (speedup 1.0000x reference)
import functools

import jax
import jax.numpy as jnp
from jax.experimental import pallas as pl
from jax.experimental.pallas import tpu as pltpu

EPS = 1e-12
BLOCK = 11264
SPLIT = 2


def _fused_body(emb_ref, t_ref, l_ref, o_ref, i_ref, u_ref, n_ref,
                wp_ref, te_ref, le_ref, oe_ref, ie_ref, ue_ref,
                nw_ref, bp_ref, nb_ref, g_ref, b_ref,
                out_ref, w_ref, tab_ref, j_ref):
    @pl.when(pl.program_id(0) == 0)
    def _prologue():
        h = wp_ref.shape[1]
        zero2 = jnp.zeros((1, h), jnp.float32)
        tab = jnp.concatenate(
            [te_ref[...], le_ref[...], oe_ref[...], ie_ref[...], ue_ref[...],
             bp_ref[...], bp_ref[...], zero2, zero2], axis=0)
        t2 = jnp.dot(tab, nw_ref[...], preferred_element_type=jnp.float32)
        si = jax.lax.broadcasted_iota(jnp.int32, (32, h), 0)
        li = jax.lax.broadcasted_iota(jnp.int32, (32, h), 1)
        brow = ((si == 28) | (si == 29)).astype(jnp.float32)
        t2 = t2 + brow * nb_ref[...]
        ind = ((si == 29) & (li == 0)).astype(jnp.float32)
        tab_ref[...] = jnp.concatenate([tab, t2, ind], axis=1)
        w2 = jnp.dot(wp_ref[...], nw_ref[...],
                     preferred_element_type=jnp.float32)
        w_ref[...] = jnp.concatenate([wp_ref[...], w2], axis=1)
        j_ref[...] = jnp.full(j_ref.shape, 1.0 / h, jnp.float32)

    one = jnp.int32(1)
    half = BLOCK // SPLIT
    for k in range(SPLIT):
        sl = pl.ds(k * half, half)
        bits = ((one << t_ref[sl]) | (one << (l_ref[sl] + 2))
                | (one << (o_ref[sl] + 10)) | (one << (i_ref[sl] + 12))
                | (one << (u_ref[sl] + 20)) | (one << (n_ref[sl] + 28)))
        iota = jax.lax.broadcasted_iota(jnp.int32, (32, half), 0)
        oh_t = (jnp.right_shift(jnp.broadcast_to(bits, (32, half)), iota)
                & 1).astype(jnp.float32)

        y0 = jnp.dot(emb_ref[sl, :], w_ref[...],
                     preferred_element_type=jnp.float32)
        y1 = jax.lax.dot_general(oh_t, tab_ref[...], (((0,), (0,)), ((), ())),
                                 preferred_element_type=jnp.float32)
        both = y0 + y1[:, 0:256]
        x = jnp.where(y1[:, 256:257] > 0.5, both[:, 128:256], both[:, 0:128])

        mu = jnp.dot(x, j_ref[...], preferred_element_type=jnp.float32)
        s2 = jnp.dot(x * x, j_ref[...], preferred_element_type=jnp.float32)
        var = s2 - mu * mu
        rstd = jax.lax.rsqrt(var + EPS)
        rg = rstd * g_ref[...]
        out_ref[sl, :] = (x - mu) * rg + b_ref[...]


@functools.partial(jax.jit, static_argnames=())
def kernel(node_embeddings, node_types, layers, operators, in_degs, out_degs,
           negs, W_proj, b_proj, type_emb, layer_emb, op_emb, in_emb, out_emb,
           neg_W, neg_b, ln_gamma, ln_beta):
    n, d = node_embeddings.shape
    h = W_proj.shape[1]
    grid = pl.cdiv(n, BLOCK)

    row = lambda i: (i, 0)
    vec = lambda i: (i,)
    rep = lambda i: (0, 0)
    ints = [node_types, layers, operators, in_degs, out_degs, negs]
    return pl.pallas_call(
        _fused_body,
        grid=(grid,),
        in_specs=[
            pl.BlockSpec((BLOCK, d), row),
            pl.BlockSpec((BLOCK,), vec),
            pl.BlockSpec((BLOCK,), vec),
            pl.BlockSpec((BLOCK,), vec),
            pl.BlockSpec((BLOCK,), vec),
            pl.BlockSpec((BLOCK,), vec),
            pl.BlockSpec((BLOCK,), vec),
            pl.BlockSpec((d, h), rep),
            pl.BlockSpec((2, h), rep),
            pl.BlockSpec((8, h), rep),
            pl.BlockSpec((2, h), rep),
            pl.BlockSpec((8, h), rep),
            pl.BlockSpec((8, h), rep),
            pl.BlockSpec((h, h), rep),
            pl.BlockSpec((1, h), rep),
            pl.BlockSpec((1, h), rep),
            pl.BlockSpec((1, h), rep),
            pl.BlockSpec((1, h), rep),
        ],
        out_specs=pl.BlockSpec((BLOCK, h), row),
        out_shape=jax.ShapeDtypeStruct((n, h), jnp.float32),
        scratch_shapes=[
            pltpu.VMEM((d, 2 * h), jnp.float32),
            pltpu.VMEM((32, 3 * h), jnp.float32),
            pltpu.VMEM((h, h), jnp.float32),
        ],
    )(node_embeddings, *ints, W_proj, type_emb, layer_emb, op_emb,
      in_emb, out_emb, neg_W, b_proj.reshape(1, h), neg_b.reshape(1, h),
      ln_gamma.reshape(1, h), ln_beta.reshape(1, h))


# speedup vs baseline: 1.1138x; 1.1138x over previous
import functools

import jax
import jax.numpy as jnp
from jax.experimental import pallas as pl
from jax.experimental.pallas import tpu as pltpu

EPS = 1e-12
BLOCK = 11264
SPLIT = 1


def _fused_body(emb_ref, t_ref, l_ref, o_ref, i_ref, u_ref, n_ref,
                wp_ref, te_ref, le_ref, oe_ref, ie_ref, ue_ref,
                nw_ref, bp_ref, nb_ref, g_ref, b_ref,
                out_ref, w_ref, tab_ref, j_ref):
    @pl.when(pl.program_id(0) == 0)
    def _prologue():
        h = wp_ref.shape[1]
        zero2 = jnp.zeros((1, h), jnp.float32)
        tab = jnp.concatenate(
            [te_ref[...], le_ref[...], oe_ref[...], ie_ref[...], ue_ref[...],
             bp_ref[...], bp_ref[...], zero2, zero2], axis=0)
        t2 = jnp.dot(tab, nw_ref[...], preferred_element_type=jnp.float32)
        si = jax.lax.broadcasted_iota(jnp.int32, (32, h), 0)
        li = jax.lax.broadcasted_iota(jnp.int32, (32, h), 1)
        brow = ((si == 28) | (si == 29)).astype(jnp.float32)
        t2 = t2 + brow * nb_ref[...]
        ind = ((si == 29) & (li == 0)).astype(jnp.float32)
        tab_ref[...] = jnp.concatenate([tab, t2, ind], axis=1)
        w2 = jnp.dot(wp_ref[...], nw_ref[...],
                     preferred_element_type=jnp.float32)
        w_ref[...] = jnp.concatenate([wp_ref[...], w2], axis=1)
        j_ref[...] = jnp.full(j_ref.shape, 1.0 / h, jnp.float32)

    one = jnp.int32(1)
    half = BLOCK // SPLIT
    for k in range(SPLIT):
        sl = pl.ds(k * half, half)
        bits = ((one << t_ref[sl]) | (one << (l_ref[sl] + 2))
                | (one << (o_ref[sl] + 10)) | (one << (i_ref[sl] + 12))
                | (one << (u_ref[sl] + 20)) | (one << (n_ref[sl] + 28)))
        iota = jax.lax.broadcasted_iota(jnp.int32, (32, half), 0)
        oh_t = (jnp.right_shift(jnp.broadcast_to(bits, (32, half)), iota)
                & 1).astype(jnp.float32)

        y0 = jnp.dot(emb_ref[sl, :], w_ref[...],
                     preferred_element_type=jnp.float32)
        y1 = jax.lax.dot_general(oh_t, tab_ref[...], (((0,), (0,)), ((), ())),
                                 preferred_element_type=jnp.float32)
        both = y0 + y1[:, 0:256]
        x = jnp.where(y1[:, 256:257] > 0.5, both[:, 128:256], both[:, 0:128])

        mu = jnp.dot(x, j_ref[...], preferred_element_type=jnp.float32)
        s2 = jnp.dot(x * x, j_ref[...], preferred_element_type=jnp.float32)
        var = s2 - mu * mu
        rstd = jax.lax.rsqrt(var + EPS)
        rg = rstd * g_ref[...]
        out_ref[sl, :] = (x - mu) * rg + b_ref[...]


@functools.partial(jax.jit, static_argnames=())
def kernel(node_embeddings, node_types, layers, operators, in_degs, out_degs,
           negs, W_proj, b_proj, type_emb, layer_emb, op_emb, in_emb, out_emb,
           neg_W, neg_b, ln_gamma, ln_beta):
    n, d = node_embeddings.shape
    h = W_proj.shape[1]
    grid = pl.cdiv(n, BLOCK)

    row = lambda i: (i, 0)
    vec = lambda i: (i,)
    rep = lambda i: (0, 0)
    ints = [node_types, layers, operators, in_degs, out_degs, negs]
    return pl.pallas_call(
        _fused_body,
        grid=(grid,),
        in_specs=[
            pl.BlockSpec((BLOCK, d), row),
            pl.BlockSpec((BLOCK,), vec),
            pl.BlockSpec((BLOCK,), vec),
            pl.BlockSpec((BLOCK,), vec),
            pl.BlockSpec((BLOCK,), vec),
            pl.BlockSpec((BLOCK,), vec),
            pl.BlockSpec((BLOCK,), vec),
            pl.BlockSpec((d, h), rep),
            pl.BlockSpec((2, h), rep),
            pl.BlockSpec((8, h), rep),
            pl.BlockSpec((2, h), rep),
            pl.BlockSpec((8, h), rep),
            pl.BlockSpec((8, h), rep),
            pl.BlockSpec((h, h), rep),
            pl.BlockSpec((1, h), rep),
            pl.BlockSpec((1, h), rep),
            pl.BlockSpec((1, h), rep),
            pl.BlockSpec((1, h), rep),
        ],
        out_specs=pl.BlockSpec((BLOCK, h), row),
        out_shape=jax.ShapeDtypeStruct((n, h), jnp.float32),
        scratch_shapes=[
            pltpu.VMEM((d, 2 * h), jnp.float32),
            pltpu.VMEM((32, 3 * h), jnp.float32),
            pltpu.VMEM((h, h), jnp.float32),
        ],
    )(node_embeddings, *ints, W_proj, type_emb, layer_emb, op_emb,
      in_emb, out_emb, neg_W, b_proj.reshape(1, h), neg_b.reshape(1, h),
      ln_gamma.reshape(1, h), ln_beta.reshape(1, h))
